# Initial kernel scaffold; baseline (speedup 1.0000x reference)
import jax
import jax.numpy as jnp
from jax.experimental import pallas as pl


def kernel(x, edge_index, edge_type, W1, root1, b1, W2, root2, b2):
    raise NotImplementedError("write your pallas kernel here")


# speedup vs baseline: 9.5911x; 9.5911x over previous
import functools

import jax
import jax.numpy as jnp
from jax import lax
from jax.experimental import pallas as pl
from jax.experimental.pallas import tpu as pltpu
from jax.experimental.pallas import tpu_sc as plsc

N = 10000
E = 320000
R = 8
D = 128
H = D // 2

NC = 2
NS = 16
NW = NC * NS

CHB = 128
CH = 79
EPT = CH * CHB
EPAD = NW * EPT

CNTSZ = 80128
ZB = CNTSZ // NS
DUMMY_KEY = 80000

NPAD = 10240
RPT = NPAD // NS

_MESH = plsc.VectorSubcoreMesh(
    core_axis_name="c", subcore_axis_name="s", num_cores=NC, num_subcores=NS)


@functools.partial(
    pl.kernel,
    out_type=jax.ShapeDtypeStruct((NC * CNTSZ,), jnp.float32),
    mesh=_MESH,
    scratch_types=[
        pltpu.VMEM((CH, CHB), jnp.int32),
        pltpu.VMEM((CHB,), jnp.float32),
        pltpu.VMEM((ZB,), jnp.float32),
        pltpu.VMEM_SHARED((CNTSZ,), jnp.float32),
    ],
    compiler_params=pltpu.CompilerParams(use_tc_tiling_on_sc=False),
)
def _count_kernel(key_hbm, cnt_hbm, keys_v, ones_v, z_v, cnt_sh):
    c = lax.axis_index("c")
    s = lax.axis_index("s")
    wid = c * NS + s

    def zbody(i, _):
        z_v[pl.ds(i * 16, 16)] = jnp.zeros((16,), jnp.float32)
        return 0
    lax.fori_loop(0, ZB // 16, zbody, 0)
    for i in range(CHB // 16):
        ones_v[pl.ds(i * 16, 16)] = jnp.ones((16,), jnp.float32)

    pltpu.sync_copy(z_v, cnt_sh.at[pl.ds(s * ZB, ZB)])
    plsc.subcore_barrier()

    pltpu.sync_copy(key_hbm.at[wid], keys_v)

    def body(j, _):
        pltpu.sync_copy(ones_v, cnt_sh.at[keys_v.at[j]], add=True)
        return 0
    lax.fori_loop(0, CH, body, 0)

    plsc.subcore_barrier()
    pltpu.sync_copy(cnt_sh.at[pl.ds(s * ZB, ZB)], z_v)
    pltpu.sync_copy(z_v, cnt_hbm.at[pl.ds(c * CNTSZ + s * ZB, ZB)])


@functools.partial(
    pl.kernel,
    out_type=jax.ShapeDtypeStruct((NW, CH, CHB), jnp.float32),
    mesh=_MESH,
    scratch_types=[
        pltpu.VMEM((CH, CHB), jnp.int32),
        pltpu.VMEM((CH, CHB), jnp.float32),
        pltpu.SemaphoreType.DMA,
    ],
    compiler_params=pltpu.CompilerParams(use_tc_tiling_on_sc=False),
)
def _scale_kernel(rcp_hbm, key_hbm, scale_hbm, keys_v, sc_acc, sem):
    c = lax.axis_index("c")
    s = lax.axis_index("s")
    wid = c * NS + s
    pltpu.sync_copy(key_hbm.at[wid], keys_v)

    def body(j, _):
        pltpu.async_copy(rcp_hbm.at[keys_v.at[j]], sc_acc.at[j], sem).wait()
        return 0
    lax.fori_loop(0, CH, body, 0)
    pltpu.sync_copy(sc_acc, scale_hbm.at[wid])


@functools.partial(
    pl.kernel,
    out_type=jax.ShapeDtypeStruct((2, NC, NPAD, H), jnp.float32),
    mesh=_MESH,
    scratch_types=[
        pltpu.VMEM((CH, CHB), jnp.int32),
        pltpu.VMEM((CH, CHB), jnp.int32),
        pltpu.VMEM((CH, CHB), jnp.float32),
        pltpu.VMEM((CHB, H), jnp.float32),
        pltpu.VMEM((CHB, H), jnp.float32),
        pltpu.VMEM_SHARED((NPAD, H), jnp.float32),
        pltpu.SemaphoreType.DMA,
    ],
    compiler_params=pltpu.CompilerParams(use_tc_tiling_on_sc=False),
)
def _agg_kernel(y0_hbm, y1_hbm, gidx_hbm, dst_hbm, scale_hbm, out_hbm,
                gidx_v, dst_v, sc_v, rows_v, z_v, agg_sh, sem):
    c = lax.axis_index("c")
    s = lax.axis_index("s")
    wid = c * NS + s

    def zbody(i, _):
        for k in range(H // 16):
            z_v[i, pl.ds(k * 16, 16)] = jnp.zeros((16,), jnp.float32)
        return 0
    lax.fori_loop(0, CHB, zbody, 0)

    pltpu.sync_copy(gidx_hbm.at[wid], gidx_v)
    pltpu.sync_copy(dst_hbm.at[wid], dst_v)
    pltpu.sync_copy(scale_hbm.at[wid], sc_v)

    for half, y_hbm in ((0, y0_hbm), (1, y1_hbm)):
        for b in range(RPT // CHB):
            pltpu.sync_copy(z_v, agg_sh.at[pl.ds(s * RPT + b * CHB, CHB)])
        plsc.subcore_barrier()

        def chunk(j, _):
            pltpu.async_copy(y_hbm.at[gidx_v.at[j]], rows_v, sem).wait()

            def edge_grp(eg, _):
                sv = sc_v[j, pl.ds(eg * 16, 16)]
                for l in range(16):
                    e = eg * 16 + l
                    sl = sv[l]
                    for k in range(H // 16):
                        rows_v[e, pl.ds(k * 16, 16)] = (
                            rows_v[e, pl.ds(k * 16, 16)] * sl)
                return 0
            lax.fori_loop(0, CHB // 16, edge_grp, 0)

            pltpu.sync_copy(rows_v, agg_sh.at[dst_v.at[j]], add=True)
            return 0
        lax.fori_loop(0, CH, chunk, 0)

        plsc.subcore_barrier()
        for b in range(RPT // CHB):
            rr = s * RPT + b * CHB
            pltpu.sync_copy(agg_sh.at[pl.ds(rr, CHB)], rows_v)
            pltpu.sync_copy(rows_v, out_hbm.at[half, c, pl.ds(rr, CHB)])
        plsc.subcore_barrier()


NB = 1000
G = N // NB


def _mm1_body(x_ref, w_ref, root_ref, b_ref, y0_ref, y1_ref, rootp_ref):
    xb = x_ref[...]
    rootp_ref[...] = (
        jnp.dot(xb, root_ref[...], preferred_element_type=jnp.float32)
        + b_ref[...])
    for r in range(R):
        yr = jnp.dot(xb, w_ref[r], preferred_element_type=jnp.float32)
        y0_ref[r] = yr[:, :H]
        y1_ref[r] = yr[:, H:]


def _mm2_body(rootp_ref, a00_ref, a01_ref, a10_ref, a11_ref,
              w_ref, root_ref, b_ref, y0_ref, y1_ref, rootp2_ref):
    agg = jnp.concatenate(
        [a00_ref[0, 0] + a01_ref[0, 0], a10_ref[0, 0] + a11_ref[0, 0]],
        axis=-1)
    h = jnp.maximum(rootp_ref[...] + agg, 0.0)
    rootp2_ref[...] = (
        jnp.dot(h, root_ref[...], preferred_element_type=jnp.float32)
        + b_ref[...])
    for r in range(R):
        yr = jnp.dot(h, w_ref[r], preferred_element_type=jnp.float32)
        y0_ref[r] = yr[:, :H]
        y1_ref[r] = yr[:, H:]


def _fin_body(rootp_ref, a00_ref, a01_ref, a10_ref, a11_ref, o_ref):
    agg = jnp.concatenate(
        [a00_ref[0, 0] + a01_ref[0, 0], a10_ref[0, 0] + a11_ref[0, 0]],
        axis=-1)
    o_ref[...] = rootp_ref[...] + agg


_W_SPEC = pl.BlockSpec((R, D, D), lambda i: (0, 0, 0))
_ROOT_SPEC = pl.BlockSpec((D, D), lambda i: (0, 0))
_B_SPEC = pl.BlockSpec((1, D), lambda i: (0, 0))
_X_SPEC = pl.BlockSpec((NB, D), lambda i: (i, 0))
_YH_SPEC = pl.BlockSpec((R, NB, H), lambda i: (0, i, 0))
_A_SPECS = [
    pl.BlockSpec((1, 1, NB, H), lambda i, _h=hh, _c=cc: (_h, _c, i, 0))
    for hh in range(2) for cc in range(NC)
]

_YH_SHAPE = jax.ShapeDtypeStruct((R, N, H), jnp.float32)
_NP_SHAPE = jax.ShapeDtypeStruct((N, D), jnp.float32)

_mm1 = pl.pallas_call(
    _mm1_body,
    grid=(G,),
    in_specs=[_X_SPEC, _W_SPEC, _ROOT_SPEC, _B_SPEC],
    out_specs=[_YH_SPEC, _YH_SPEC, _X_SPEC],
    out_shape=[_YH_SHAPE, _YH_SHAPE, _NP_SHAPE],
)

_mm2 = pl.pallas_call(
    _mm2_body,
    grid=(G,),
    in_specs=[_X_SPEC] + _A_SPECS + [_W_SPEC, _ROOT_SPEC, _B_SPEC],
    out_specs=[_YH_SPEC, _YH_SPEC, _X_SPEC],
    out_shape=[_YH_SHAPE, _YH_SHAPE, _NP_SHAPE],
)

_fin = pl.pallas_call(
    _fin_body,
    grid=(G,),
    in_specs=[_X_SPEC] + _A_SPECS,
    out_specs=_X_SPEC,
    out_shape=jax.ShapeDtypeStruct((N, D), jnp.float32),
)


def kernel(x, edge_index, edge_type, W1, root1, b1, W2, root2, b2):
    src = edge_index[0].astype(jnp.int32)
    dst = edge_index[1].astype(jnp.int32)
    et = edge_type.astype(jnp.int32)

    pad = EPAD - E
    gidx = jnp.concatenate([et * N + src, jnp.zeros((pad,), jnp.int32)])
    key = jnp.concatenate([et * N + dst, jnp.full((pad,), DUMMY_KEY, jnp.int32)])
    dstp = jnp.concatenate([dst, jnp.zeros((pad,), jnp.int32)])

    gidx = gidx.reshape(NW, CH, CHB)
    key = key.reshape(NW, CH, CHB)
    dstp = dstp.reshape(NW, CH, CHB)

    cnt2 = _count_kernel(key).reshape(NC, CNTSZ)
    cnt = cnt2[0] + cnt2[1]
    rcp = 1.0 / jnp.clip(cnt, 1.0, None)
    rcp = jnp.where(jnp.arange(CNTSZ) < R * N, rcp, 0.0)
    scale = _scale_kernel(rcp, key)

    b1r = b1.reshape(1, D)
    b2r = b2.reshape(1, D)

    y10, y11, rootp1 = _mm1(x, W1, root1, b1r)
    a1 = _agg_kernel(y10.reshape(R * N, H), y11.reshape(R * N, H),
                     gidx, dstp, scale)

    y20, y21, rootp2 = _mm2(rootp1, a1, a1, a1, a1, W2, root2, b2r)
    a2 = _agg_kernel(y20.reshape(R * N, H), y21.reshape(R * N, H),
                     gidx, dstp, scale)

    return _fin(rootp2, a2, a2, a2, a2)


# speedup vs baseline: 12.6090x; 1.3147x over previous
import functools

import jax
import jax.numpy as jnp
from jax import lax
from jax.experimental import pallas as pl
from jax.experimental.pallas import tpu as pltpu
from jax.experimental.pallas import tpu_sc as plsc

N = 10000
E = 320000
R = 8
D = 128
H = D // 2

NC = 2
NS = 16
NW = NC * NS

CHB = 128
CH = 80
EPT = CH * CHB
EPAD = NW * EPT

CNTSZ = 80128
ZB = CNTSZ // NS
DUMMY_KEY = 80000

NPAD = 10240
RPT = NPAD // NS

_MESH = plsc.VectorSubcoreMesh(
    core_axis_name="c", subcore_axis_name="s", num_cores=NC, num_subcores=NS)


@functools.partial(
    pl.kernel,
    out_type=jax.ShapeDtypeStruct((NC * CNTSZ,), jnp.float32),
    mesh=_MESH,
    scratch_types=[
        pltpu.VMEM((CH, CHB), jnp.int32),
        pltpu.VMEM((CHB,), jnp.float32),
        pltpu.VMEM((ZB,), jnp.float32),
        pltpu.VMEM_SHARED((CNTSZ,), jnp.float32),
    ],
    compiler_params=pltpu.CompilerParams(use_tc_tiling_on_sc=False),
)
def _count_kernel(key_hbm, cnt_hbm, keys_v, ones_v, z_v, cnt_sh):
    c = lax.axis_index("c")
    s = lax.axis_index("s")
    wid = c * NS + s

    def zbody(i, _):
        z_v[pl.ds(i * 16, 16)] = jnp.zeros((16,), jnp.float32)
        return 0
    lax.fori_loop(0, ZB // 16, zbody, 0)
    for i in range(CHB // 16):
        ones_v[pl.ds(i * 16, 16)] = jnp.ones((16,), jnp.float32)

    pltpu.sync_copy(z_v, cnt_sh.at[pl.ds(s * ZB, ZB)])
    plsc.subcore_barrier()

    pltpu.sync_copy(key_hbm.at[wid], keys_v)

    def body(j, _):
        pltpu.sync_copy(ones_v, cnt_sh.at[keys_v.at[j]], add=True)
        return 0
    lax.fori_loop(0, CH, body, 0)

    plsc.subcore_barrier()
    pltpu.sync_copy(cnt_sh.at[pl.ds(s * ZB, ZB)], z_v)
    pltpu.sync_copy(z_v, cnt_hbm.at[pl.ds(c * CNTSZ + s * ZB, ZB)])


@functools.partial(
    pl.kernel,
    out_type=jax.ShapeDtypeStruct((NW, CH, CHB), jnp.float32),
    mesh=_MESH,
    scratch_types=[
        pltpu.VMEM((CH, CHB), jnp.int32),
        pltpu.VMEM((CH, CHB), jnp.float32),
        pltpu.SemaphoreType.DMA,
    ],
    compiler_params=pltpu.CompilerParams(use_tc_tiling_on_sc=False),
)
def _scale_kernel(rcp_hbm, key_hbm, scale_hbm, keys_v, sc_acc, sem):
    c = lax.axis_index("c")
    s = lax.axis_index("s")
    wid = c * NS + s
    pltpu.sync_copy(key_hbm.at[wid], keys_v)

    def body(q, _):
        j0 = q * 8
        for b in range(8):
            pltpu.make_async_copy(
                rcp_hbm.at[keys_v.at[j0 + b]], sc_acc.at[j0 + b], sem).start()
        for b in range(8):
            pltpu.make_async_copy(
                rcp_hbm.at[keys_v.at[j0 + b]], sc_acc.at[j0 + b], sem).wait()
        return 0
    lax.fori_loop(0, CH // 8, body, 0)
    pltpu.sync_copy(sc_acc, scale_hbm.at[wid])


@functools.partial(
    pl.kernel,
    out_type=jax.ShapeDtypeStruct((2, NC, NPAD, H), jnp.float32),
    mesh=_MESH,
    scratch_types=[
        pltpu.VMEM((CH, CHB), jnp.int32),
        pltpu.VMEM((CH, CHB), jnp.int32),
        pltpu.VMEM((CH, CHB), jnp.float32),
        pltpu.VMEM((2, CHB, H), jnp.float32),
        pltpu.VMEM((2, CHB, H), jnp.float32),
        pltpu.VMEM((CHB, H), jnp.float32),
        pltpu.VMEM_SHARED((NPAD, H), jnp.float32),
        pltpu.SemaphoreType.DMA((2,)),
        pltpu.SemaphoreType.DMA((2,)),
    ],
    compiler_params=pltpu.CompilerParams(use_tc_tiling_on_sc=False),
)
def _agg_kernel(y0_hbm, y1_hbm, gidx_hbm, dst_hbm, scale_hbm, out_hbm,
                gidx_v, dst_v, sc_v, grows, srows, z_v, agg_sh, gsem, ssem):
    c = lax.axis_index("c")
    s = lax.axis_index("s")
    wid = c * NS + s

    def zbody(i, _):
        for k in range(H // 16):
            z_v[i, pl.ds(k * 16, 16)] = jnp.zeros((16,), jnp.float32)
        return 0
    lax.fori_loop(0, CHB, zbody, 0)

    pltpu.sync_copy(gidx_hbm.at[wid], gidx_v)
    pltpu.sync_copy(dst_hbm.at[wid], dst_v)
    pltpu.sync_copy(scale_hbm.at[wid], sc_v)

    for half, y_hbm in ((0, y0_hbm), (1, y1_hbm)):
        for b in range(RPT // CHB):
            pltpu.sync_copy(z_v, agg_sh.at[pl.ds(s * RPT + b * CHB, CHB)])
        plsc.subcore_barrier()

        for b in (0, 1):
            pltpu.make_async_copy(
                y_hbm.at[gidx_v.at[b]], grows.at[b], gsem.at[b]).start()

        def pair(q, _):
            j2 = q * 2
            for b in (0, 1):
                j = j2 + b
                pltpu.make_async_copy(
                    y_hbm.at[gidx_v.at[j]], grows.at[b], gsem.at[b]).wait()

                @pl.when(q > 0)
                def _():
                    pltpu.make_async_copy(
                        srows.at[b], agg_sh.at[dst_v.at[j - 2]],
                        ssem.at[b]).wait()

                def edge_grp(eg, _):
                    sv = sc_v[j, pl.ds(eg * 16, 16)]
                    for l in range(16):
                        e = eg * 16 + l
                        sl = sv[l]
                        for k in range(H // 16):
                            srows[b, e, pl.ds(k * 16, 16)] = (
                                grows[b, e, pl.ds(k * 16, 16)] * sl)
                    return 0
                lax.fori_loop(0, CHB // 16, edge_grp, 0)

                pltpu.make_async_copy(
                    srows.at[b], agg_sh.at[dst_v.at[j]],
                    ssem.at[b]).start(add=True)

                @pl.when(q < CH // 2 - 1)
                def _():
                    pltpu.make_async_copy(
                        y_hbm.at[gidx_v.at[j + 2]], grows.at[b],
                        gsem.at[b]).start()
            return 0
        lax.fori_loop(0, CH // 2, pair, 0)

        for b in (0, 1):
            pltpu.make_async_copy(
                srows.at[b], agg_sh.at[dst_v.at[CH - 2 + b]],
                ssem.at[b]).wait()

        plsc.subcore_barrier()
        for b in range(RPT // CHB):
            rr = s * RPT + b * CHB
            pltpu.sync_copy(agg_sh.at[pl.ds(rr, CHB)], z_v)
            pltpu.sync_copy(z_v, out_hbm.at[half, c, pl.ds(rr, CHB)])
        plsc.subcore_barrier()
        if half == 0:
            def rezero(i, _):
                for k in range(H // 16):
                    z_v[i, pl.ds(k * 16, 16)] = jnp.zeros((16,), jnp.float32)
                return 0
            lax.fori_loop(0, CHB, rezero, 0)


NB = 1000
G = N // NB


def _mm1_body(x_ref, w_ref, root_ref, b_ref, y0_ref, y1_ref, rootp_ref):
    xb = x_ref[...]
    rootp_ref[...] = (
        jnp.dot(xb, root_ref[...], preferred_element_type=jnp.float32)
        + b_ref[...])
    for r in range(R):
        yr = jnp.dot(xb, w_ref[r], preferred_element_type=jnp.float32)
        y0_ref[r] = yr[:, :H]
        y1_ref[r] = yr[:, H:]


def _mm2_body(rootp_ref, a00_ref, a01_ref, a10_ref, a11_ref,
              w_ref, root_ref, b_ref, y0_ref, y1_ref, rootp2_ref):
    agg = jnp.concatenate(
        [a00_ref[0, 0] + a01_ref[0, 0], a10_ref[0, 0] + a11_ref[0, 0]],
        axis=-1)
    h = jnp.maximum(rootp_ref[...] + agg, 0.0)
    rootp2_ref[...] = (
        jnp.dot(h, root_ref[...], preferred_element_type=jnp.float32)
        + b_ref[...])
    for r in range(R):
        yr = jnp.dot(h, w_ref[r], preferred_element_type=jnp.float32)
        y0_ref[r] = yr[:, :H]
        y1_ref[r] = yr[:, H:]


def _fin_body(rootp_ref, a00_ref, a01_ref, a10_ref, a11_ref, o_ref):
    agg = jnp.concatenate(
        [a00_ref[0, 0] + a01_ref[0, 0], a10_ref[0, 0] + a11_ref[0, 0]],
        axis=-1)
    o_ref[...] = rootp_ref[...] + agg


_W_SPEC = pl.BlockSpec((R, D, D), lambda i: (0, 0, 0))
_ROOT_SPEC = pl.BlockSpec((D, D), lambda i: (0, 0))
_B_SPEC = pl.BlockSpec((1, D), lambda i: (0, 0))
_X_SPEC = pl.BlockSpec((NB, D), lambda i: (i, 0))
_YH_SPEC = pl.BlockSpec((R, NB, H), lambda i: (0, i, 0))
_A_SPECS = [
    pl.BlockSpec((1, 1, NB, H), lambda i, _h=hh, _c=cc: (_h, _c, i, 0))
    for hh in range(2) for cc in range(NC)
]

_YH_SHAPE = jax.ShapeDtypeStruct((R, N, H), jnp.float32)
_NP_SHAPE = jax.ShapeDtypeStruct((N, D), jnp.float32)

_mm1 = pl.pallas_call(
    _mm1_body,
    grid=(G,),
    in_specs=[_X_SPEC, _W_SPEC, _ROOT_SPEC, _B_SPEC],
    out_specs=[_YH_SPEC, _YH_SPEC, _X_SPEC],
    out_shape=[_YH_SHAPE, _YH_SHAPE, _NP_SHAPE],
)

_mm2 = pl.pallas_call(
    _mm2_body,
    grid=(G,),
    in_specs=[_X_SPEC] + _A_SPECS + [_W_SPEC, _ROOT_SPEC, _B_SPEC],
    out_specs=[_YH_SPEC, _YH_SPEC, _X_SPEC],
    out_shape=[_YH_SHAPE, _YH_SHAPE, _NP_SHAPE],
)

_fin = pl.pallas_call(
    _fin_body,
    grid=(G,),
    in_specs=[_X_SPEC] + _A_SPECS,
    out_specs=_X_SPEC,
    out_shape=jax.ShapeDtypeStruct((N, D), jnp.float32),
)


def kernel(x, edge_index, edge_type, W1, root1, b1, W2, root2, b2):
    src = edge_index[0].astype(jnp.int32)
    dst = edge_index[1].astype(jnp.int32)
    et = edge_type.astype(jnp.int32)

    pad = EPAD - E
    gidx = jnp.concatenate([et * N + src, jnp.zeros((pad,), jnp.int32)])
    key = jnp.concatenate([et * N + dst, jnp.full((pad,), DUMMY_KEY, jnp.int32)])
    dstp = jnp.concatenate([dst, jnp.zeros((pad,), jnp.int32)])

    gidx = gidx.reshape(NW, CH, CHB)
    key = key.reshape(NW, CH, CHB)
    dstp = dstp.reshape(NW, CH, CHB)

    cnt2 = _count_kernel(key).reshape(NC, CNTSZ)
    cnt = cnt2[0] + cnt2[1]
    rcp = 1.0 / jnp.clip(cnt, 1.0, None)
    rcp = jnp.where(jnp.arange(CNTSZ) < R * N, rcp, 0.0)
    scale = _scale_kernel(rcp, key)

    b1r = b1.reshape(1, D)
    b2r = b2.reshape(1, D)

    y10, y11, rootp1 = _mm1(x, W1, root1, b1r)
    a1 = _agg_kernel(y10.reshape(R * N, H), y11.reshape(R * N, H),
                     gidx, dstp, scale)

    y20, y21, rootp2 = _mm2(rootp1, a1, a1, a1, a1, W2, root2, b2r)
    a2 = _agg_kernel(y20.reshape(R * N, H), y21.reshape(R * N, H),
                     gidx, dstp, scale)

    return _fin(rootp2, a2, a2, a2, a2)
